# baseline (speedup 1.0000x reference)
import functools
import math

import numpy as np
import jax
import jax.numpy as jnp
from jax import lax
from jax.experimental import pallas as pl
from jax.experimental.pallas import tpu as pltpu
from jax.experimental.pallas import tpu_sc as plsc


@functools.lru_cache(maxsize=None)
def _pos_encoding(seq_len: int, d_model: int):
    pos = np.arange(seq_len, dtype=np.float32)[:, None]
    div = np.exp(
        np.arange(0, d_model, 2, dtype=np.float32) * (-np.log(10000.0) / d_model)
    )
    pe = np.zeros((seq_len, d_model), dtype=np.float32)
    pe[:, 0::2] = np.sin(pos * div)
    pe[:, 1::2] = np.cos(pos * div)
    return jnp.asarray(pe)


@functools.lru_cache(maxsize=None)
def _build(n_rows: int, seq_len: int, vocab: int, d_model: int):
    info = plsc.get_sparse_core_info()
    num_cores, num_subcores, lanes = info.num_cores, info.num_subcores, info.num_lanes
    num_workers = num_cores * num_subcores
    per_worker = n_rows // num_workers
    chunk = 64
    if per_worker % chunk:
        chunk = per_worker
    n_chunks = per_worker // chunk
    segs = d_model // lanes
    scale = jnp.float32(math.sqrt(d_model))
    mesh = plsc.VectorSubcoreMesh(core_axis_name="c", subcore_axis_name="s")

    @functools.partial(
        pl.kernel,
        mesh=mesh,
        out_type=[
            jax.ShapeDtypeStruct((n_rows, d_model), jnp.float32),
            jax.ShapeDtypeStruct((n_rows, d_model), jnp.float32),
        ],
        scratch_types=[
            pltpu.VMEM((chunk,), jnp.int32),
            pltpu.VMEM((chunk, d_model), jnp.float32),
            pltpu.VMEM((chunk, d_model), jnp.float32),
            pltpu.SemaphoreType.DMA,
            pltpu.SemaphoreType.DMA,
        ],
    )
    def k(src_hbm, tgt_hbm, enc_hbm, dec_hbm, pe_hbm,
          src_out, tgt_out, idx_v, gbuf, obuf, gsem, psem):
        wid = lax.axis_index("s") * num_cores + lax.axis_index("c")
        base = wid * per_worker
        for t in range(2):
            idx_hbm = (src_hbm, tgt_hbm)[t]
            table = (enc_hbm, dec_hbm)[t]
            out = (src_out, tgt_out)[t]
            for c in range(n_chunks):
                row0 = base + c * chunk
                pe0 = lax.rem(row0, seq_len)
                pltpu.sync_copy(idx_hbm.at[pl.ds(row0, chunk)], idx_v)
                gcp = pltpu.async_copy(table.at[idx_v], gbuf, gsem)
                pcp = pltpu.async_copy(pe_hbm.at[pl.ds(pe0, chunk)], obuf, psem)
                gcp.wait()
                pcp.wait()

                def body(r, carry):
                    for j in range(segs):
                        v = gbuf[r, pl.ds(j * lanes, lanes)] * scale
                        plsc.addupdate(obuf.at[r, pl.ds(j * lanes, lanes)], v)
                    return carry

                lax.fori_loop(0, chunk, body, 0)
                pltpu.sync_copy(obuf, out.at[pl.ds(row0, chunk)])

    return k


def kernel(src, tgt, src_mask, tgt_mask, enc_table, dec_table):
    batch, seq = src.shape
    vocab, d_model = enc_table.shape
    pe = _pos_encoding(seq, d_model)
    k = _build(batch * seq, seq, vocab, d_model)
    src_e, tgt_e = k(src.reshape(-1), tgt.reshape(-1), enc_table, dec_table, pe)
    return (
        src_e.reshape(batch, seq, d_model),
        tgt_e.reshape(batch, seq, d_model),
    )
